# baseline (speedup 1.0000x reference)
import functools

import jax
import jax.numpy as jnp
from jax import lax
from jax.experimental import pallas as pl
from jax.experimental.pallas import tpu as pltpu
from jax.experimental.pallas import tpu_sc as plsc

N = 10000
NP = 10240
E = 160000
EP = 163840
NREL = 4
D = 256
QD = 64
NT = 16
ROWS_PT = EP // NT // 128
CNT_BINS = NREL * NP + 512
CNT_PT = CNT_BINS // NT
ACC_PT = NP // NT
BN = 1024


def _tc_body(x_ref, w_ref, r_ref, b_ref, z_ref, base_ref):
    xb = x_ref[...]
    z_ref[...] = jnp.dot(xb, w_ref[...], preferred_element_type=jnp.float32)
    bfull = jnp.dot(xb, r_ref[...], preferred_element_type=jnp.float32) + b_ref[...]
    for cc in range(2):
        for hh in range(2):
            o = (cc * 2 + hh) * QD
            base_ref[cc, hh] = bfull[:, o:o + QD]


def _tc_matmuls(xp, wcat, root, bias2d):
    return pl.pallas_call(
        _tc_body,
        grid=(NP // BN,),
        in_specs=[
            pl.BlockSpec((BN, D), lambda i: (i, 0)),
            pl.BlockSpec((D, NREL * D), lambda i: (0, 0)),
            pl.BlockSpec((D, D), lambda i: (0, 0)),
            pl.BlockSpec((1, D), lambda i: (0, 0)),
        ],
        out_specs=[
            pl.BlockSpec((BN, NREL * D), lambda i: (i, 0)),
            pl.BlockSpec((2, 2, BN, QD), lambda i: (0, 0, i, 0)),
        ],
        out_shape=[
            jax.ShapeDtypeStruct((NP, NREL * D), jnp.float32),
            jax.ShapeDtypeStruct((2, 2, NP, QD), jnp.float32),
        ],
    )(xp, wcat, root, bias2d)


_SC_MESH = plsc.VectorSubcoreMesh(core_axis_name="c", subcore_axis_name="s")


@functools.partial(
    pl.kernel,
    out_type=jax.ShapeDtypeStruct((2, 2, NP, QD), jnp.float32),
    mesh=_SC_MESH,
    scratch_types=[
        pltpu.VMEM((ROWS_PT, 128), jnp.int32),
        pltpu.VMEM((ROWS_PT + 1, 128), jnp.int32),
        pltpu.VMEM((ROWS_PT, 128), jnp.int32),
        pltpu.VMEM((ROWS_PT, 128), jnp.float32),
        pltpu.VMEM((4, 128), jnp.float32),
        pltpu.VMEM((128,), jnp.int32),
        pltpu.VMEM((128,), jnp.float32),
        pltpu.VMEM((128, 128), jnp.float32),
        pltpu.VMEM((128, QD), jnp.float32),
        pltpu.VMEM((128, QD), jnp.float32),
        pltpu.VMEM((CNT_BINS // NT,), jnp.float32),
        pltpu.VMEM_SHARED((CNT_BINS,), jnp.float32),
        pltpu.VMEM_SHARED((NP, QD), jnp.float32),
        pltpu.SemaphoreType.DMA,
        pltpu.SemaphoreType.DMA,
        pltpu.SemaphoreType.DMA,
        pltpu.SemaphoreType.DMA,
    ],
    compiler_params=pltpu.CompilerParams(use_tc_tiling_on_sc=False),
)
def _sc_scatter(key_hbm, gidx_hbm, dst_hbm, w_hbm, z2_hbm, base_hbm, out_hbm,
                keyb, gidxb, dstb, wb, cntrow, idx1d, ones, rows_v, rows_s0,
                rows_s1, zbuf, cnt_sh, acc_sh, sem, sems0, sems1, semh):
    c = lax.axis_index("c")
    s = lax.axis_index("s")
    r0 = s * ROWS_PT
    pltpu.sync_copy(key_hbm.at[pl.ds(r0, ROWS_PT)], keyb)
    pltpu.sync_copy(gidx_hbm.at[pl.ds(r0, ROWS_PT)],
                    gidxb.at[pl.ds(0, ROWS_PT)])
    pltpu.sync_copy(dst_hbm.at[pl.ds(r0, ROWS_PT)], dstb)
    pltpu.sync_copy(w_hbm.at[pl.ds(r0, ROWS_PT)], wb)
    for i in range(CNT_PT // 16):
        zbuf[pl.ds(i * 16, 16)] = jnp.zeros((16,), jnp.float32)
    pltpu.sync_copy(zbuf, cnt_sh.at[pl.ds(s * CNT_PT, CNT_PT)])
    for i in range(8):
        ones[pl.ds(i * 16, 16)] = jnp.ones((16,), jnp.float32)
        gidxb[ROWS_PT, pl.ds(i * 16, 16)] = jnp.zeros((16,), jnp.int32)
    plsc.subcore_barrier()

    def hist_body(wv, carry):
        cps = [pltpu.async_copy(ones, cnt_sh.at[keyb.at[wv * 4 + i]], sem,
                                add=True) for i in range(4)]
        for cp in cps:
            cp.wait()
        return carry
    lax.fori_loop(0, ROWS_PT // 4, hist_body, 0)
    plsc.subcore_barrier()

    def scale_body(wv, carry):
        cps = [pltpu.async_copy(cnt_sh.at[keyb.at[wv * 4 + i]], cntrow.at[i],
                                sem) for i in range(4)]
        for cp in cps:
            cp.wait()
        for i in range(4):
            j = wv * 4 + i
            for k in range(8):
                sl = pl.ds(k * 16, 16)
                wb[j, sl] = wb[j, sl] / jnp.maximum(cntrow[i, sl], 1.0)
                gidxb[j, sl] = gidxb[j, sl] + c
        return carry
    lax.fori_loop(0, ROWS_PT // 4, scale_body, 0)

    for h in range(2):
        pltpu.sync_copy(base_hbm.at[c].at[h].at[pl.ds(s * ACC_PT, ACC_PT)],
                        acc_sh.at[pl.ds(s * ACC_PT, ACC_PT)])
        plsc.subcore_barrier()

        pltpu.async_copy(z2_hbm.at[gidxb.at[0].at[pl.ds(0, 64)]],
                         rows_v.at[pl.ds(0, 64)], sem)
        pltpu.async_copy(z2_hbm.at[gidxb.at[0].at[pl.ds(64, 64)]],
                         rows_v.at[pl.ds(64, 64)], semh)

        def half_chunk(j, t, rows_s, sems):
            pltpu.make_async_copy(z2_hbm.at[gidxb.at[j].at[pl.ds(0, 64)]],
                                  rows_v.at[pl.ds(0, 64)], sem).wait()

            @pl.when(t > 0)
            def _():
                pltpu.make_async_copy(rows_s, acc_sh.at[dstb.at[j]],
                                      sems).wait()
            for g in range(4):
                sv = wb[j, pl.ds(g * 16, 16)]
                for i in range(16):
                    spl = jnp.broadcast_to(sv[i:i + 1], (16,))
                    rv = rows_v.at[g * 16 + i]
                    rs = rows_s.at[g * 16 + i]
                    for k in range(QD // 16):
                        rs[pl.ds(k * 16, 16)] = rv[pl.ds(h * QD + k * 16, 16)] * spl
            pltpu.make_async_copy(z2_hbm.at[gidxb.at[j].at[pl.ds(64, 64)]],
                                  rows_v.at[pl.ds(64, 64)], semh).wait()
            for g in range(4, 8):
                sv = wb[j, pl.ds(g * 16, 16)]
                for i in range(16):
                    spl = jnp.broadcast_to(sv[i:i + 1], (16,))
                    rv = rows_v.at[g * 16 + i]
                    rs = rows_s.at[g * 16 + i]
                    for k in range(QD // 16):
                        rs[pl.ds(k * 16, 16)] = rv[pl.ds(h * QD + k * 16, 16)] * spl
            pltpu.async_copy(z2_hbm.at[gidxb.at[j + 1].at[pl.ds(0, 64)]],
                             rows_v.at[pl.ds(0, 64)], sem)
            pltpu.async_copy(z2_hbm.at[gidxb.at[j + 1].at[pl.ds(64, 64)]],
                             rows_v.at[pl.ds(64, 64)], semh)
            pltpu.async_copy(rows_s, acc_sh.at[dstb.at[j]], sems, add=True)

        def pair_body(t, carry):
            half_chunk(2 * t, t, rows_s0, sems0)
            half_chunk(2 * t + 1, t, rows_s1, sems1)
            return carry
        lax.fori_loop(0, ROWS_PT // 2, pair_body, 0)
        pltpu.make_async_copy(rows_s0, acc_sh.at[dstb.at[ROWS_PT - 2]],
                              sems0).wait()
        pltpu.make_async_copy(rows_s1, acc_sh.at[dstb.at[ROWS_PT - 1]],
                              sems1).wait()
        pltpu.make_async_copy(z2_hbm.at[gidxb.at[ROWS_PT].at[pl.ds(0, 64)]],
                              rows_v.at[pl.ds(0, 64)], sem).wait()
        pltpu.make_async_copy(z2_hbm.at[gidxb.at[ROWS_PT].at[pl.ds(64, 64)]],
                              rows_v.at[pl.ds(64, 64)], semh).wait()
        plsc.subcore_barrier()
        pltpu.sync_copy(acc_sh.at[pl.ds(s * ACC_PT, ACC_PT)],
                        out_hbm.at[c].at[h].at[pl.ds(s * ACC_PT, ACC_PT)])
        plsc.subcore_barrier()


def kernel(x, edge_index, edge_type, edge_weight, weight, root, bias):
    src = edge_index[0].astype(jnp.int32)
    dst = edge_index[1].astype(jnp.int32)
    rel = edge_type.astype(jnp.int32)
    w = edge_weight.astype(jnp.float32)
    pad = EP - E
    ar = jnp.arange(pad, dtype=jnp.int32)
    gidx = jnp.concatenate([src * 8 + rel * 2, (ar % 2048) * 8]).reshape(-1, 128)
    key = jnp.concatenate([dst * NREL + rel,
                           NREL * NP + (ar % 512)]).reshape(-1, 128)
    dstp = jnp.concatenate([dst, ar % 4096]).reshape(-1, 128)
    wp = jnp.concatenate([w, jnp.zeros((pad,), jnp.float32)]).reshape(-1, 128)

    xp = jnp.pad(x, ((0, NP - N), (0, 0)))
    wcat = weight.transpose(1, 0, 2).reshape(D, NREL * D)
    z, base_q = _tc_matmuls(xp, wcat, root, bias.reshape(1, D))
    z2 = z.reshape(NP * 8, 2 * QD)
    out_q = _sc_scatter(key, gidx, dstp, wp, z2, base_q)
    return out_q.transpose(2, 0, 1, 3).reshape(NP, D)[:N]


# speedup vs baseline: 1.0007x; 1.0007x over previous
import functools

import jax
import jax.numpy as jnp
from jax import lax
from jax.experimental import pallas as pl
from jax.experimental.pallas import tpu as pltpu
from jax.experimental.pallas import tpu_sc as plsc

N = 10000
NP = 10240
E = 160000
EP = 163840
NREL = 4
D = 256
QD = 64
NT = 16
ROWS_PT = EP // NT // 128
CNT_BINS = NREL * NP + 512
CNT_PT = CNT_BINS // NT
ACC_PT = NP // NT
BN = 1024


def _tc_body(x_ref, w_ref, r_ref, b_ref, z_ref, base_ref):
    xb = x_ref[...]
    z_ref[...] = jnp.dot(xb, w_ref[...], preferred_element_type=jnp.float32)
    bfull = jnp.dot(xb, r_ref[...], preferred_element_type=jnp.float32) + b_ref[...]
    for cc in range(2):
        for hh in range(2):
            o = (cc * 2 + hh) * QD
            base_ref[cc, hh] = bfull[:, o:o + QD]


def _tc_matmuls(xp, wcat, root, bias2d):
    return pl.pallas_call(
        _tc_body,
        grid=(NP // BN,),
        in_specs=[
            pl.BlockSpec((BN, D), lambda i: (i, 0)),
            pl.BlockSpec((D, NREL * D), lambda i: (0, 0)),
            pl.BlockSpec((D, D), lambda i: (0, 0)),
            pl.BlockSpec((1, D), lambda i: (0, 0)),
        ],
        out_specs=[
            pl.BlockSpec((BN, NREL * D), lambda i: (i, 0)),
            pl.BlockSpec((2, 2, BN, QD), lambda i: (0, 0, i, 0)),
        ],
        out_shape=[
            jax.ShapeDtypeStruct((NP, NREL * D), jnp.float32),
            jax.ShapeDtypeStruct((2, 2, NP, QD), jnp.float32),
        ],
    )(xp, wcat, root, bias2d)


_SC_MESH = plsc.VectorSubcoreMesh(core_axis_name="c", subcore_axis_name="s")


@functools.partial(
    pl.kernel,
    out_type=jax.ShapeDtypeStruct((2, 2, NP, QD), jnp.float32),
    mesh=_SC_MESH,
    scratch_types=[
        pltpu.VMEM((ROWS_PT, 128), jnp.int32),
        pltpu.VMEM((ROWS_PT + 1, 128), jnp.int32),
        pltpu.VMEM((ROWS_PT, 128), jnp.int32),
        pltpu.VMEM((ROWS_PT, 128), jnp.float32),
        pltpu.VMEM((4, 128), jnp.float32),
        pltpu.VMEM((128,), jnp.int32),
        pltpu.VMEM((128,), jnp.float32),
        pltpu.VMEM((128, 128), jnp.float32),
        pltpu.VMEM((128, QD), jnp.float32),
        pltpu.VMEM((128, QD), jnp.float32),
        pltpu.VMEM((CNT_BINS // NT,), jnp.float32),
        pltpu.VMEM_SHARED((CNT_BINS,), jnp.float32),
        pltpu.VMEM_SHARED((NP, QD), jnp.float32),
        pltpu.SemaphoreType.DMA,
        pltpu.SemaphoreType.DMA,
        pltpu.SemaphoreType.DMA,
        pltpu.SemaphoreType.DMA,
    ],
    compiler_params=pltpu.CompilerParams(use_tc_tiling_on_sc=False),
)
def _sc_scatter(key_hbm, gidx_hbm, dst_hbm, w_hbm, z2_hbm, base_hbm, out_hbm,
                keyb, gidxb, dstb, wb, cntrow, idx1d, ones, rows_v, rows_s0,
                rows_s1, zbuf, cnt_sh, acc_sh, sem, sems0, sems1, semh):
    c = lax.axis_index("c")
    s = lax.axis_index("s")
    r0 = s * ROWS_PT
    pltpu.sync_copy(key_hbm.at[pl.ds(r0, ROWS_PT)], keyb)
    pltpu.sync_copy(gidx_hbm.at[pl.ds(r0, ROWS_PT)],
                    gidxb.at[pl.ds(0, ROWS_PT)])
    pltpu.sync_copy(dst_hbm.at[pl.ds(r0, ROWS_PT)], dstb)
    pltpu.sync_copy(w_hbm.at[pl.ds(r0, ROWS_PT)], wb)
    for i in range(CNT_PT // 16):
        zbuf[pl.ds(i * 16, 16)] = jnp.zeros((16,), jnp.float32)
    pltpu.sync_copy(zbuf, cnt_sh.at[pl.ds(s * CNT_PT, CNT_PT)])
    for i in range(8):
        ones[pl.ds(i * 16, 16)] = jnp.ones((16,), jnp.float32)
        gidxb[ROWS_PT, pl.ds(i * 16, 16)] = jnp.zeros((16,), jnp.int32)
    plsc.subcore_barrier()

    def hist_body(wv, carry):
        cps = [pltpu.async_copy(ones, cnt_sh.at[keyb.at[wv * 4 + i]], sem,
                                add=True) for i in range(4)]
        for cp in cps:
            cp.wait()
        return carry
    lax.fori_loop(0, ROWS_PT // 4, hist_body, 0)
    plsc.subcore_barrier()

    def scale_body(wv, carry):
        cps = [pltpu.async_copy(cnt_sh.at[keyb.at[wv * 4 + i]], cntrow.at[i],
                                sem) for i in range(4)]
        for cp in cps:
            cp.wait()
        for i in range(4):
            j = wv * 4 + i
            for k in range(8):
                sl = pl.ds(k * 16, 16)
                wb[j, sl] = wb[j, sl] / jnp.maximum(cntrow[i, sl], 1.0)
                gidxb[j, sl] = gidxb[j, sl] + c
        return carry
    lax.fori_loop(0, ROWS_PT // 4, scale_body, 0)

    for h in range(2):
        pltpu.sync_copy(base_hbm.at[c].at[h].at[pl.ds(s * ACC_PT, ACC_PT)],
                        acc_sh.at[pl.ds(s * ACC_PT, ACC_PT)])
        plsc.subcore_barrier()

        pltpu.async_copy(z2_hbm.at[gidxb.at[0].at[pl.ds(0, 64)]],
                         rows_v.at[pl.ds(0, 64)], sem)
        pltpu.async_copy(z2_hbm.at[gidxb.at[0].at[pl.ds(64, 64)]],
                         rows_v.at[pl.ds(64, 64)], semh)

        def half_chunk(j, t, rows_s, sems):
            pltpu.make_async_copy(z2_hbm.at[gidxb.at[j].at[pl.ds(0, 64)]],
                                  rows_v.at[pl.ds(0, 64)], sem).wait()

            @pl.when(t > 0)
            def _():
                pltpu.make_async_copy(rows_s, acc_sh.at[dstb.at[j]],
                                      sems).wait()
            for g in range(4):
                sv = wb[j, pl.ds(g * 16, 16)]
                for i in range(16):
                    spl = jnp.broadcast_to(sv[i:i + 1], (16,))
                    rv = rows_v.at[g * 16 + i]
                    rs = rows_s.at[g * 16 + i]
                    for k in range(QD // 16):
                        rs[pl.ds(k * 16, 16)] = rv[pl.ds(h * QD + k * 16, 16)] * spl
            pltpu.make_async_copy(z2_hbm.at[gidxb.at[j].at[pl.ds(64, 64)]],
                                  rows_v.at[pl.ds(64, 64)], semh).wait()
            for g in range(4, 8):
                sv = wb[j, pl.ds(g * 16, 16)]
                for i in range(16):
                    spl = jnp.broadcast_to(sv[i:i + 1], (16,))
                    rv = rows_v.at[g * 16 + i]
                    rs = rows_s.at[g * 16 + i]
                    for k in range(QD // 16):
                        rs[pl.ds(k * 16, 16)] = rv[pl.ds(h * QD + k * 16, 16)] * spl
            pltpu.async_copy(rows_s, acc_sh.at[dstb.at[j]], sems, add=True)
            pltpu.async_copy(z2_hbm.at[gidxb.at[j + 1].at[pl.ds(0, 64)]],
                             rows_v.at[pl.ds(0, 64)], sem)
            pltpu.async_copy(z2_hbm.at[gidxb.at[j + 1].at[pl.ds(64, 64)]],
                             rows_v.at[pl.ds(64, 64)], semh)

        def pair_body(t, carry):
            half_chunk(2 * t, t, rows_s0, sems0)
            half_chunk(2 * t + 1, t, rows_s1, sems1)
            return carry
        lax.fori_loop(0, ROWS_PT // 2, pair_body, 0)
        pltpu.make_async_copy(rows_s0, acc_sh.at[dstb.at[ROWS_PT - 2]],
                              sems0).wait()
        pltpu.make_async_copy(rows_s1, acc_sh.at[dstb.at[ROWS_PT - 1]],
                              sems1).wait()
        pltpu.make_async_copy(z2_hbm.at[gidxb.at[ROWS_PT].at[pl.ds(0, 64)]],
                              rows_v.at[pl.ds(0, 64)], sem).wait()
        pltpu.make_async_copy(z2_hbm.at[gidxb.at[ROWS_PT].at[pl.ds(64, 64)]],
                              rows_v.at[pl.ds(64, 64)], semh).wait()
        plsc.subcore_barrier()
        pltpu.sync_copy(acc_sh.at[pl.ds(s * ACC_PT, ACC_PT)],
                        out_hbm.at[c].at[h].at[pl.ds(s * ACC_PT, ACC_PT)])
        plsc.subcore_barrier()


def kernel(x, edge_index, edge_type, edge_weight, weight, root, bias):
    src = edge_index[0].astype(jnp.int32)
    dst = edge_index[1].astype(jnp.int32)
    rel = edge_type.astype(jnp.int32)
    w = edge_weight.astype(jnp.float32)
    pad = EP - E
    ar = jnp.arange(pad, dtype=jnp.int32)
    gidx = jnp.concatenate([src * 8 + rel * 2, (ar % 2048) * 8]).reshape(-1, 128)
    key = jnp.concatenate([dst * NREL + rel,
                           NREL * NP + (ar % 512)]).reshape(-1, 128)
    dstp = jnp.concatenate([dst, ar % 4096]).reshape(-1, 128)
    wp = jnp.concatenate([w, jnp.zeros((pad,), jnp.float32)]).reshape(-1, 128)

    xp = jnp.pad(x, ((0, NP - N), (0, 0)))
    wcat = weight.transpose(1, 0, 2).reshape(D, NREL * D)
    z, base_q = _tc_matmuls(xp, wcat, root, bias.reshape(1, D))
    z2 = z.reshape(NP * 8, 2 * QD)
    out_q = _sc_scatter(key, gidx, dstp, wp, z2, base_q)
    return out_q.transpose(2, 0, 1, 3).reshape(NP, D)[:N]


# speedup vs baseline: 1.6759x; 1.6747x over previous
import functools

import jax
import jax.numpy as jnp
from jax import lax
from jax.experimental import pallas as pl
from jax.experimental.pallas import tpu as pltpu
from jax.experimental.pallas import tpu_sc as plsc

N = 10000
NP = 10240
E = 160000
EP = 163840
NREL = 4
D = 256
QD = 64
NT = 16
ROWS_PT = EP // NT // 128
CNT_BINS = NREL * NP + 512
CNT_PT = CNT_BINS // NT
ACC_PT = NP // NT
BN = 1024


def _tc_body(x_ref, w_ref, r_ref, b_ref, z_ref, base_ref):
    xb = x_ref[...]
    z_ref[...] = jnp.dot(xb, w_ref[...], preferred_element_type=jnp.float32)
    bfull = jnp.dot(xb, r_ref[...], preferred_element_type=jnp.float32) + b_ref[...]
    for cc in range(2):
        for hh in range(2):
            o = (cc * 2 + hh) * QD
            base_ref[cc, hh] = bfull[:, o:o + QD]


def _tc_matmuls(xp, wcat, root, bias2d):
    return pl.pallas_call(
        _tc_body,
        grid=(NP // BN,),
        in_specs=[
            pl.BlockSpec((BN, D), lambda i: (i, 0)),
            pl.BlockSpec((D, NREL * D), lambda i: (0, 0)),
            pl.BlockSpec((D, D), lambda i: (0, 0)),
            pl.BlockSpec((1, D), lambda i: (0, 0)),
        ],
        out_specs=[
            pl.BlockSpec((BN, NREL * D), lambda i: (i, 0)),
            pl.BlockSpec((2, 2, BN, QD), lambda i: (0, 0, i, 0)),
        ],
        out_shape=[
            jax.ShapeDtypeStruct((NP, NREL * D), jnp.float32),
            jax.ShapeDtypeStruct((2, 2, NP, QD), jnp.float32),
        ],
    )(xp, wcat, root, bias2d)


_SC_MESH = plsc.VectorSubcoreMesh(core_axis_name="c", subcore_axis_name="s")


@functools.partial(
    pl.kernel,
    out_type=jax.ShapeDtypeStruct((2, 2, NP, QD), jnp.float32),
    mesh=_SC_MESH,
    scratch_types=[
        pltpu.VMEM((ROWS_PT, 128), jnp.int32),
        pltpu.VMEM((ROWS_PT, 128), jnp.int32),
        pltpu.VMEM((ROWS_PT, 128), jnp.int32),
        pltpu.VMEM((ROWS_PT, 128), jnp.float32),
        pltpu.VMEM((4, 128), jnp.float32),
        pltpu.VMEM((128,), jnp.int32),
        pltpu.VMEM((128,), jnp.float32),
        pltpu.VMEM((128, 128), jnp.float32),
        pltpu.VMEM((128, QD), jnp.float32),
        pltpu.VMEM((128, QD), jnp.float32),
        pltpu.VMEM((CNT_BINS // NT,), jnp.float32),
        pltpu.VMEM_SHARED((CNT_BINS,), jnp.float32),
        pltpu.VMEM_SHARED((NP, QD), jnp.float32),
        pltpu.SemaphoreType.DMA,
        pltpu.SemaphoreType.DMA,
        pltpu.SemaphoreType.DMA,
        pltpu.SemaphoreType.DMA,
    ],
    compiler_params=pltpu.CompilerParams(use_tc_tiling_on_sc=False),
)
def _sc_scatter(key_hbm, gidx_hbm, dst_hbm, w_hbm, z2_hbm, base_hbm, out_hbm,
                keyb, gidxb, dstb, wb, cntrow, idx1d, ones, rows_v, rows_s0,
                rows_s1, zbuf, cnt_sh, acc_sh, sem, sems0, sems1, semh):
    c = lax.axis_index("c")
    s = lax.axis_index("s")
    r0 = s * ROWS_PT
    pltpu.sync_copy(key_hbm.at[pl.ds(r0, ROWS_PT)], keyb)
    pltpu.sync_copy(gidx_hbm.at[pl.ds(r0, ROWS_PT)], gidxb)
    pltpu.sync_copy(dst_hbm.at[pl.ds(r0, ROWS_PT)], dstb)
    pltpu.sync_copy(w_hbm.at[pl.ds(r0, ROWS_PT)], wb)
    for i in range(CNT_PT // 16):
        zbuf[pl.ds(i * 16, 16)] = jnp.zeros((16,), jnp.float32)
    pltpu.sync_copy(zbuf, cnt_sh.at[pl.ds(s * CNT_PT, CNT_PT)])
    for i in range(8):
        ones[pl.ds(i * 16, 16)] = jnp.ones((16,), jnp.float32)
    plsc.subcore_barrier()

    def hist_body(wv, carry):
        cps = [pltpu.async_copy(ones, cnt_sh.at[keyb.at[wv * 4 + i]], sem,
                                add=True) for i in range(4)]
        for cp in cps:
            cp.wait()
        return carry
    lax.fori_loop(0, ROWS_PT // 4, hist_body, 0)
    plsc.subcore_barrier()

    def scale_body(wv, carry):
        cps = [pltpu.async_copy(cnt_sh.at[keyb.at[wv * 4 + i]], cntrow.at[i],
                                sem) for i in range(4)]
        for cp in cps:
            cp.wait()
        for i in range(4):
            j = wv * 4 + i
            for k in range(8):
                sl = pl.ds(k * 16, 16)
                wb[j, sl] = wb[j, sl] / jnp.maximum(cntrow[i, sl], 1.0)
                gidxb[j, sl] = gidxb[j, sl] + c
        return carry
    lax.fori_loop(0, ROWS_PT // 4, scale_body, 0)

    for h in range(2):
        pltpu.sync_copy(base_hbm.at[c].at[h].at[pl.ds(s * ACC_PT, ACC_PT)],
                        acc_sh.at[pl.ds(s * ACC_PT, ACC_PT)])
        plsc.subcore_barrier()

        def half_chunk(j, t, rows_s, sems):
            pltpu.async_copy(z2_hbm.at[gidxb.at[j].at[pl.ds(0, 64)]],
                             rows_v.at[pl.ds(0, 64)], sem)
            pltpu.async_copy(z2_hbm.at[gidxb.at[j].at[pl.ds(64, 64)]],
                             rows_v.at[pl.ds(64, 64)], semh)
            pltpu.make_async_copy(z2_hbm.at[gidxb.at[j].at[pl.ds(0, 64)]],
                                  rows_v.at[pl.ds(0, 64)], sem).wait()

            @pl.when(t > 0)
            def _():
                pltpu.make_async_copy(rows_s, acc_sh.at[dstb.at[j]],
                                      sems).wait()
            for g in range(4):
                sv = wb[j, pl.ds(g * 16, 16)]
                for i in range(16):
                    spl = jnp.broadcast_to(sv[i:i + 1], (16,))
                    rv = rows_v.at[g * 16 + i]
                    rs = rows_s.at[g * 16 + i]
                    for k in range(QD // 16):
                        rs[pl.ds(k * 16, 16)] = rv[pl.ds(h * QD + k * 16, 16)] * spl
            pltpu.make_async_copy(z2_hbm.at[gidxb.at[j].at[pl.ds(64, 64)]],
                                  rows_v.at[pl.ds(64, 64)], semh).wait()
            for g in range(4, 8):
                sv = wb[j, pl.ds(g * 16, 16)]
                for i in range(16):
                    spl = jnp.broadcast_to(sv[i:i + 1], (16,))
                    rv = rows_v.at[g * 16 + i]
                    rs = rows_s.at[g * 16 + i]
                    for k in range(QD // 16):
                        rs[pl.ds(k * 16, 16)] = rv[pl.ds(h * QD + k * 16, 16)] * spl
            pltpu.async_copy(rows_s, acc_sh.at[dstb.at[j]], sems, add=True)

        def pair_body(t, carry):
            half_chunk(2 * t, t, rows_s0, sems0)
            half_chunk(2 * t + 1, t, rows_s1, sems1)
            return carry
        lax.fori_loop(0, ROWS_PT // 2, pair_body, 0)
        pltpu.make_async_copy(rows_s0, acc_sh.at[dstb.at[ROWS_PT - 2]],
                              sems0).wait()
        pltpu.make_async_copy(rows_s1, acc_sh.at[dstb.at[ROWS_PT - 1]],
                              sems1).wait()
        plsc.subcore_barrier()
        pltpu.sync_copy(acc_sh.at[pl.ds(s * ACC_PT, ACC_PT)],
                        out_hbm.at[c].at[h].at[pl.ds(s * ACC_PT, ACC_PT)])
        plsc.subcore_barrier()


def kernel(x, edge_index, edge_type, edge_weight, weight, root, bias):
    src = edge_index[0].astype(jnp.int32)
    dst = edge_index[1].astype(jnp.int32)
    rel = edge_type.astype(jnp.int32)
    w = edge_weight.astype(jnp.float32)
    pad = EP - E
    ar = jnp.arange(pad, dtype=jnp.int32)
    gidx = jnp.concatenate([src * 8 + rel * 2, (ar % 2048) * 8]).reshape(-1, 128)
    key = jnp.concatenate([dst * NREL + rel,
                           NREL * NP + (ar % 512)]).reshape(-1, 128)
    dstp = jnp.concatenate([dst, ar % 4096]).reshape(-1, 128)
    wp = jnp.concatenate([w, jnp.zeros((pad,), jnp.float32)]).reshape(-1, 128)

    xp = jnp.pad(x, ((0, NP - N), (0, 0)))
    wcat = weight.transpose(1, 0, 2).reshape(D, NREL * D)
    z, base_q = _tc_matmuls(xp, wcat, root, bias.reshape(1, D))
    z2 = z.reshape(NP * 8, 2 * QD)
    out_q = _sc_scatter(key, gidx, dstp, wp, z2, base_q)
    return out_q.transpose(2, 0, 1, 3).reshape(NP, D)[:N]
